# baseline (speedup 1.0000x reference)
import functools

import jax
import jax.numpy as jnp
from jax import lax
from jax.experimental import pallas as pl
from jax.experimental.pallas import tpu as pltpu
from jax.experimental.pallas import tpu_sc as plsc

_S = 8192
_N = 4096
_N_SC = 3584
_N_TC = _N - _N_SC
_C0 = 128
_CHUNK = 128
_NW = 32
_ROWS = _N_SC // _NW


def _prep_kernel(ifb_ref, iast_ref, exp_u_ref, tle_ref, bnd_ref, r_ref,
                 t_ref, th_ref):
    r = r_ref[0, 0]
    tle = tle_ref[0, 0]
    bnd = bnd_ref[0, 0]
    bounds = jnp.max(jnp.sum(ifb_ref[...], axis=-1)) * 5.0
    sr = bounds * r

    iast = iast_ref[...]

    uc = jnp.clip(exp_u_ref[...], 0.0, 1.0 - 1e-7)
    e2 = -jnp.log1p(-uc) / sr
    i0 = lax.broadcasted_iota(jnp.int32, (128, 128), 0)
    i1 = lax.broadcasted_iota(jnp.int32, (128, 128), 1)
    upper = (i0 <= i1).astype(jnp.float32)
    cums = lax.dot(e2, upper, precision=lax.Precision.HIGHEST,
                   preferred_element_type=jnp.float32)
    totals = cums[:, 127:128]
    j0 = lax.broadcasted_iota(jnp.int32, (64, 64), 0)
    j1 = lax.broadcasted_iota(jnp.int32, (64, 64), 1)
    strict = (j1 < j0).astype(jnp.float32)
    offs = lax.dot(strict, totals, precision=lax.Precision.HIGHEST,
                   preferred_element_type=jnp.float32)
    t2 = cums + offs + tle

    t_last = t2[63, 127]
    base = jnp.where(t_last > bnd, t_last, bnd)
    t_ref[0:64, :] = t2
    t_ref[64:72, :] = jnp.full((8, 128), base)

    k0 = lax.broadcasted_iota(jnp.int32, (1024, 128), 0)
    k1 = lax.broadcasted_iota(jnp.int32, (1024, 128), 1)
    sel = ((k0 >> 3) == k1).astype(jnp.float32)
    ti = lax.dot(iast, sel, precision=lax.Precision.HIGHEST,
                 preferred_element_type=jnp.float32)
    th_ref[...] = ti * (r / sr)


def _scalarize(x):
    return x if x.ndim == 0 else x[0]


def _tc_scan(u_ref, t_ref, th_ref, rst_ref, w_ref):
    t72 = t_ref[...]
    big = t72[63, 127] + 1.0
    base = t72[64, 0]

    def chunk(c, acc):
        uc = u_ref[:, pl.ds(c * 128, 128)]
        th_c = th_ref[pl.ds(c, 1), :]
        t_c = t_ref[pl.ds(c, 1), :]
        return jnp.minimum(acc, jnp.where(uc < th_c, t_c, big))

    acc = lax.fori_loop(0, _S // 128, chunk,
                        jnp.full((128, 128), big, jnp.float32))
    m = jnp.min(acc, axis=1, keepdims=True)
    rst_ref[...] = jnp.where(m < big, m, base)
    w_ref[...] = jnp.full((128, 1), 1.0 / _N, jnp.float32)


def _sc_scan(th_hbm, t_hbm, u_hbm, rst_hbm, w_hbm,
             th_v, t_v, ub_v, urow_v, rst_v, done_v, sem):
    wid = lax.axis_index("s") * 2 + lax.axis_index("c")
    base_row = wid * _ROWS

    c1 = pltpu.async_copy(th_hbm, th_v, sem)
    c2 = pltpu.async_copy(t_hbm, t_v, sem)
    c3 = pltpu.async_copy(
        u_hbm.at[pl.ds(base_row, _ROWS), pl.ds(0, _C0)], ub_v, sem)
    c1.wait()
    c2.wait()
    c3.wait()

    lanes = lax.broadcasted_iota(jnp.int32, (16,), 0)
    base_splat = jnp.full((16,), _scalarize(t_v[pl.ds(_S, 16)]))

    def group_body(g, und):
        rowids = g * 16 + lanes

        def cond(c):
            cc, alldone = c[0], c[1]
            return jnp.logical_and(jnp.logical_not(alldone), cc < _C0)

        def body(c):
            cc, _, done, colsel = c
            thch = th_v[pl.ds(cc, 16)]
            for s in range(8):
                col = cc + s
                u_c = plsc.load_gather(
                    ub_v, [rowids, jnp.full((16,), col, jnp.int32)])
                th_c = jnp.full((16,), thch[s])
                mask = u_c < th_c
                newly = jnp.logical_and(mask, jnp.logical_not(done))
                colsel = jnp.where(
                    newly, jnp.full((16,), col, jnp.int32), colsel)
                done = jnp.logical_or(done, mask)
            nd = _scalarize(plsc.all_reduce_population_count(done))
            return (cc + 8, nd == 16, done, colsel)

        _, _, done, colsel = lax.while_loop(
            cond, body,
            (jnp.int32(0), jnp.bool_(False),
             jnp.zeros((16,), jnp.bool_), jnp.zeros((16,), jnp.int32)))

        times = plsc.load_gather(t_v, [colsel])
        rst_v[pl.ds(g * 16, 16)] = jnp.where(done, times, base_splat)
        done_v[pl.ds(g * 16, 16)] = done.astype(jnp.int32)
        nd = _scalarize(plsc.all_reduce_population_count(done))
        return und + (16 - nd)

    und = lax.fori_loop(0, _ROWS // 16, group_body, jnp.int32(0))

    def scan_chunks(j0, j1, sel0, load_u):
        def cond(c):
            j, found = c[0], c[1]
            return jnp.logical_and(jnp.logical_not(found), j < j1)

        def body(c):
            j, _, sel = c
            u16 = load_u(j)
            th16 = th_v[pl.ds(j * 16, 16)]
            ffs = _scalarize(plsc.all_reduce_ffs(u16 < th16))
            found = ffs < 16
            sel = jnp.where(found, j * 16 + ffs, sel)
            return (j + 1, found, sel)

        _, found, sel = lax.while_loop(
            cond, body, (j0, jnp.bool_(False), sel0))
        return found, sel

    @pl.when(und > 0)
    def _():
        def row_body(r, carry):
            fnd = _scalarize(plsc.load_gather(
                done_v, [jnp.full((16,), r, jnp.int32)]))

            @pl.when(fnd == 0)
            def _():
                def fb_cond(c):
                    k, found = c[0], c[1]
                    return jnp.logical_and(
                        jnp.logical_not(found), k < _S // _CHUNK)

                def fb_body(c):
                    k, _, sel_in = c
                    pltpu.async_copy(
                        u_hbm.at[base_row + r, pl.ds(k * _CHUNK, _CHUNK)],
                        urow_v, sem).wait()

                    def load_fb(j):
                        return urow_v[pl.ds((j - k * (_CHUNK // 16)) * 16, 16)]

                    found, sel = scan_chunks(
                        k * (_CHUNK // 16), (k + 1) * (_CHUNK // 16),
                        sel_in, load_fb)
                    return (k + 1, found, sel)

                _, found, sel = lax.while_loop(
                    fb_cond, fb_body,
                    (jnp.int32(_C0 // _CHUNK), jnp.bool_(False), jnp.int32(0)))

                @pl.when(found)
                def _():
                    val = _scalarize(t_v[pl.ds(sel, 16)])
                    plsc.store_scatter(
                        rst_v, [jnp.full((16,), r, jnp.int32)],
                        jnp.full((16,), val), mask=lanes == 0)

            return carry

        lax.fori_loop(0, _ROWS, row_body, jnp.int32(0))

    pltpu.async_copy(rst_v, rst_hbm.at[pl.ds(base_row, _ROWS)], sem).wait()
    w = jnp.full((16,), 1.0 / _N, jnp.float32)
    for g in range(_ROWS // 16):
        rst_v[pl.ds(g * 16, 16)] = w
    pltpu.async_copy(rst_v, w_hbm.at[pl.ds(base_row, _ROWS)], sem).wait()


def kernel(intensities_for_bound, intensities_at_sampled_times, exp_u,
           unif_numbers, time_last_event, boundary, ratio):
    num_sample, S = unif_numbers.shape
    tle = time_last_event.reshape(1, 1)
    bnd = boundary.reshape(1, 1)
    r = ratio.reshape(1, 1)

    t72, th64 = pl.pallas_call(
        _prep_kernel,
        out_shape=(
            jax.ShapeDtypeStruct((72, 128), jnp.float32),
            jax.ShapeDtypeStruct((64, 128), jnp.float32),
        ),
    )(intensities_for_bound,
      intensities_at_sampled_times.reshape(64, 1024),
      exp_u.reshape(64, 128), tle, bnd, r)

    mesh = plsc.VectorSubcoreMesh(core_axis_name="c", subcore_axis_name="s")
    sck = functools.partial(
        pl.kernel,
        mesh=mesh,
        compiler_params=pltpu.CompilerParams(needs_layout_passes=False),
        out_type=(
            jax.ShapeDtypeStruct((_N_SC,), jnp.float32),
            jax.ShapeDtypeStruct((_N_SC,), jnp.float32),
        ),
        scratch_types=[
            pltpu.VMEM((S,), jnp.float32),
            pltpu.VMEM((72 * 128,), jnp.float32),
            pltpu.VMEM((_ROWS, _C0), jnp.float32),
            pltpu.VMEM((_CHUNK,), jnp.float32),
            pltpu.VMEM((_ROWS,), jnp.float32),
            pltpu.VMEM((_ROWS,), jnp.int32),
            pltpu.SemaphoreType.DMA,
        ],
    )(_sc_scan)
    rst_sc, w_sc = sck(th64.reshape(S), t72.reshape(72 * 128), unif_numbers)

    rst_tc, w_tc = pl.pallas_call(
        _tc_scan,
        grid=(_N_TC // 128,),
        in_specs=[
            pl.BlockSpec((128, S), lambda i: (i + _N_SC // 128, 0)),
            pl.BlockSpec((72, 128), lambda i: (0, 0)),
            pl.BlockSpec((64, 128), lambda i: (0, 0)),
        ],
        out_specs=(
            pl.BlockSpec((128, 1), lambda i: (i, 0)),
            pl.BlockSpec((128, 1), lambda i: (i, 0)),
        ),
        out_shape=(
            jax.ShapeDtypeStruct((_N_TC, 1), jnp.float32),
            jax.ShapeDtypeStruct((_N_TC, 1), jnp.float32),
        ),
    )(unif_numbers, t72, th64)

    rst = jnp.concatenate([rst_sc, rst_tc.reshape(_N_TC)])
    w = jnp.concatenate([w_sc, w_tc.reshape(_N_TC)])
    return (rst, w)


# speedup vs baseline: 1.2688x; 1.2688x over previous
import functools

import jax
import jax.numpy as jnp
from jax import lax
from jax.experimental import pallas as pl
from jax.experimental.pallas import tpu as pltpu
from jax.experimental.pallas import tpu_sc as plsc

_S = 8192
_N = 4096
_C0 = 128
_CV = _C0
_CHUNK = 128
_NW = 32
_ROWS = _N // _NW
_UBSTRIDE = _C0 + 1


def _prep_kernel(ifb_ref, iast_ref, exp_u_ref, tle_ref, bnd_ref, r_ref,
                 t_ref, th_ref):
    r = r_ref[0, 0]
    tle = tle_ref[0, 0]
    bnd = bnd_ref[0, 0]
    bounds = jnp.max(jnp.sum(ifb_ref[...], axis=-1)) * 5.0
    sr = bounds * r

    iast = iast_ref[...]

    uc = jnp.clip(exp_u_ref[...], 0.0, 1.0 - 1e-7)
    e2 = -jnp.log1p(-uc) / sr
    i0 = lax.broadcasted_iota(jnp.int32, (128, 128), 0)
    i1 = lax.broadcasted_iota(jnp.int32, (128, 128), 1)
    upper = (i0 <= i1).astype(jnp.float32)
    cums = lax.dot(e2, upper, precision=lax.Precision.HIGHEST,
                   preferred_element_type=jnp.float32)
    totals = cums[:, 127:128]
    j0 = lax.broadcasted_iota(jnp.int32, (64, 64), 0)
    j1 = lax.broadcasted_iota(jnp.int32, (64, 64), 1)
    strict = (j1 < j0).astype(jnp.float32)
    offs = lax.dot(strict, totals, precision=lax.Precision.HIGHEST,
                   preferred_element_type=jnp.float32)
    t2 = cums + offs + tle

    t_last = t2[63, 127]
    base = jnp.where(t_last > bnd, t_last, bnd)
    t_ref[0:64, :] = t2
    t_ref[64:72, :] = jnp.full((8, 128), base)

    k0 = lax.broadcasted_iota(jnp.int32, (1024, 128), 0)
    k1 = lax.broadcasted_iota(jnp.int32, (1024, 128), 1)
    sel = ((k0 >> 3) == k1).astype(jnp.float32)
    ti = lax.dot(iast, sel, precision=lax.Precision.HIGHEST,
                 preferred_element_type=jnp.float32)
    th_ref[...] = ti * (r / sr)


def _scalarize(x):
    return x if x.ndim == 0 else x[0]


def _sc_scan(th_hbm, t_hbm, u_hbm, rst_hbm, w_hbm,
             th_v, t_v, ub_v, urow_v, thch_v, tch_v, rst_v, done_v, sem):
    wid = lax.axis_index("s") * 2 + lax.axis_index("c")
    base_row = wid * _ROWS

    c1 = pltpu.async_copy(th_hbm.at[pl.ds(0, _C0)], th_v.at[pl.ds(0, _C0)], sem)
    c2 = pltpu.async_copy(t_hbm.at[pl.ds(0, _C0)], t_v.at[pl.ds(0, _C0)], sem)
    c2b = pltpu.async_copy(t_hbm.at[pl.ds(_S, 16)], t_v.at[pl.ds(_C0, 16)], sem)
    c3 = pltpu.async_copy(
        u_hbm.at[pl.ds(base_row, _ROWS), pl.ds(0, _C0)], ub_v, sem)
    c1.wait()
    c2.wait()
    c2b.wait()
    c3.wait()

    lanes = lax.broadcasted_iota(jnp.int32, (16,), 0)
    base_splat = jnp.full((16,), _scalarize(t_v[pl.ds(_C0, 16)]))

    def group_body(g, und):
        rowids = g * 16 + lanes

        def cond(c):
            cc, alldone = c[0], c[1]
            return jnp.logical_and(jnp.logical_not(alldone), cc < _CV)

        def body(c):
            cc, _, done, colsel = c
            thch = th_v[pl.ds(cc, 16)]
            for s in range(8):
                col = cc + s
                u_c = plsc.load_gather(
                    ub_v, [rowids, jnp.full((16,), col, jnp.int32)])
                th_c = jnp.full((16,), thch[s])
                mask = u_c < th_c
                newly = jnp.logical_and(mask, jnp.logical_not(done))
                colsel = jnp.where(
                    newly, jnp.full((16,), col, jnp.int32), colsel)
                done = jnp.logical_or(done, mask)
            nd = _scalarize(plsc.all_reduce_population_count(done))
            return (cc + 8, nd == 16, done, colsel)

        _, _, done, colsel = lax.while_loop(
            cond, body,
            (jnp.int32(0), jnp.bool_(False),
             jnp.zeros((16,), jnp.bool_), jnp.zeros((16,), jnp.int32)))

        times = plsc.load_gather(t_v, [colsel])
        rst_v[pl.ds(g * 16, 16)] = jnp.where(done, times, base_splat)
        done_v[pl.ds(g * 16, 16)] = done.astype(jnp.int32)
        nd = _scalarize(plsc.all_reduce_population_count(done))
        return und + (16 - nd)

    und = lax.fori_loop(0, _ROWS // 16, group_body, jnp.int32(0))

    def scan_chunks(j0, j1, sel0, load_u, load_th):
        def cond(c):
            j, found = c[0], c[1]
            return jnp.logical_and(jnp.logical_not(found), j < j1)

        def body(c):
            j, _, sel = c
            u16 = load_u(j)
            th16 = load_th(j)
            ffs = _scalarize(plsc.all_reduce_ffs(u16 < th16))
            found = ffs < 16
            sel = jnp.where(found, j * 16 + ffs, sel)
            return (j + 1, found, sel)

        _, found, sel = lax.while_loop(
            cond, body, (j0, jnp.bool_(False), sel0))
        return found, sel

    @pl.when(und > 0)
    def _():
        def row_body(r, carry):
            fnd = _scalarize(plsc.load_gather(
                done_v, [jnp.full((16,), r, jnp.int32)]))

            @pl.when(fnd == 0)
            def _():
                def fb_cond(c):
                    k, found = c[0], c[1]
                    return jnp.logical_and(
                        jnp.logical_not(found), k < _S // _CHUNK)

                def fb_body(c):
                    k, _, sel_in = c
                    cu = pltpu.async_copy(
                        u_hbm.at[base_row + r, pl.ds(k * _CHUNK, _CHUNK)],
                        urow_v, sem)
                    cth = pltpu.async_copy(
                        th_hbm.at[pl.ds(k * _CHUNK, _CHUNK)], thch_v, sem)
                    cu.wait()
                    cth.wait()

                    def load_fb(j):
                        return urow_v[pl.ds((j - k * (_CHUNK // 16)) * 16, 16)]

                    def load_fb_th(j):
                        return thch_v[pl.ds((j - k * (_CHUNK // 16)) * 16, 16)]

                    found, sel = scan_chunks(
                        k * (_CHUNK // 16), (k + 1) * (_CHUNK // 16),
                        sel_in, load_fb, load_fb_th)
                    return (k + 1, found, sel)

                _, found, sel = lax.while_loop(
                    fb_cond, fb_body,
                    (jnp.int32(_CV // _CHUNK), jnp.bool_(False), jnp.int32(0)))

                @pl.when(found)
                def _():
                    sal = pl.multiple_of(jnp.bitwise_and(sel, jnp.int32(-8)), 8)
                    pltpu.async_copy(
                        t_hbm.at[pl.ds(sal, 16)], tch_v, sem).wait()
                    val = _scalarize(plsc.load_gather(
                        tch_v, [jnp.full((16,), sel - sal, jnp.int32)]))
                    plsc.store_scatter(
                        rst_v, [jnp.full((16,), r, jnp.int32)],
                        jnp.full((16,), val), mask=lanes == 0)

            return carry

        lax.fori_loop(0, _ROWS, row_body, jnp.int32(0))

    pltpu.async_copy(rst_v, rst_hbm.at[pl.ds(base_row, _ROWS)], sem).wait()
    w = jnp.full((16,), 1.0 / _N, jnp.float32)
    for g in range(_ROWS // 16):
        rst_v[pl.ds(g * 16, 16)] = w
    pltpu.async_copy(rst_v, w_hbm.at[pl.ds(base_row, _ROWS)], sem).wait()


def kernel(intensities_for_bound, intensities_at_sampled_times, exp_u,
           unif_numbers, time_last_event, boundary, ratio):
    num_sample, S = unif_numbers.shape
    tle = time_last_event.reshape(1, 1)
    bnd = boundary.reshape(1, 1)
    r = ratio.reshape(1, 1)

    t72, th64 = pl.pallas_call(
        _prep_kernel,
        out_shape=(
            jax.ShapeDtypeStruct((72, 128), jnp.float32),
            jax.ShapeDtypeStruct((64, 128), jnp.float32),
        ),
    )(intensities_for_bound,
      intensities_at_sampled_times.reshape(64, 1024),
      exp_u.reshape(64, 128), tle, bnd, r)

    mesh = plsc.VectorSubcoreMesh(core_axis_name="c", subcore_axis_name="s")
    sck = functools.partial(
        pl.kernel,
        mesh=mesh,
        compiler_params=pltpu.CompilerParams(needs_layout_passes=False),
        out_type=(
            jax.ShapeDtypeStruct((num_sample,), jnp.float32),
            jax.ShapeDtypeStruct((num_sample,), jnp.float32),
        ),
        scratch_types=[
            pltpu.VMEM((_C0 + 16,), jnp.float32),
            pltpu.VMEM((_C0 + 16,), jnp.float32),
            pltpu.VMEM((_ROWS, _C0), jnp.float32),
            pltpu.VMEM((_CHUNK,), jnp.float32),
            pltpu.VMEM((_CHUNK,), jnp.float32),
            pltpu.VMEM((16,), jnp.float32),
            pltpu.VMEM((_ROWS,), jnp.float32),
            pltpu.VMEM((_ROWS,), jnp.int32),
            pltpu.SemaphoreType.DMA,
        ],
    )(_sc_scan)
    rst, w = sck(th64.reshape(S), t72.reshape(72 * 128), unif_numbers)
    return (rst, w)
